# Initial kernel scaffold; baseline (speedup 1.0000x reference)
import jax
import jax.numpy as jnp
from jax.experimental import pallas as pl


def kernel(x, edge_idx, W1, b1, g1, be1, W2, b2, g2, be2, W3, b3):
    raise NotImplementedError("write your pallas kernel here")


# speedup vs baseline: 14.7338x; 14.7338x over previous
import functools

import jax
import jax.numpy as jnp
from jax import lax
from jax.experimental import pallas as pl
from jax.experimental.pallas import tpu as pltpu
from jax.experimental.pallas import tpu_sc as plsc

N = 10000
E = 320000
F = 128
F3 = 40
F3P = 48

NC = 2
NS = 16
NW = NC * NS

CHUNK = 128
NCH = 79
EPT = NCH * CHUNK
EPAD = NW * EPT
RPT = 632
ZROWS = 32
NP = NS * RPT

_f32 = jnp.float32
_i32 = jnp.int32

_MESH = plsc.VectorSubcoreMesh(core_axis_name="c", subcore_axis_name="s")


@functools.partial(
    pl.kernel,
    mesh=_MESH,
    out_type=jax.ShapeDtypeStruct((2 * NC * NP,), _f32),
    scratch_types=[
        pltpu.VMEM((CHUNK,), _i32),
        pltpu.VMEM((CHUNK,), _i32),
        pltpu.VMEM((CHUNK,), _f32),
        pltpu.VMEM((CHUNK,), _f32),
        pltpu.VMEM((RPT,), _f32),
        pltpu.VMEM_SHARED((NP,), _f32),
        pltpu.VMEM_SHARED((NP,), _f32),
    ],
)
def _hist(row_hbm, col_hbm, out_hbm, rowv, colv, vew, vsf, zb, aew, asf):
    cid = lax.axis_index("c")
    sid = lax.axis_index("s")
    wid = sid * NC + cid

    z16 = jnp.zeros((16,), _f32)

    def zb_body(g, _):
        zb[pl.ds(g * 16, 16)] = z16
        return 0

    lax.fori_loop(0, RPT // 16, zb_body, 0)
    zb[pl.ds(RPT - 16, 16)] = z16
    pltpu.sync_copy(zb, aew.at[pl.ds(sid * RPT, RPT)])
    pltpu.sync_copy(zb, asf.at[pl.ds(sid * RPT, RPT)])
    plsc.subcore_barrier()

    def chunk(j, _):
        base = wid * EPT + j * CHUNK
        pltpu.sync_copy(row_hbm.at[pl.ds(base, CHUNK)], rowv)
        pltpu.sync_copy(col_hbm.at[pl.ds(base, CHUNK)], colv)

        def grp(g, _):
            r = rowv[pl.ds(g * 16, 16)]
            c = colv[pl.ds(g * 16, 16)]
            ew = jnp.where(r != c, 1.0, 0.0).astype(_f32)
            vew[pl.ds(g * 16, 16)] = ew
            vsf[pl.ds(g * 16, 16)] = 1.0 - ew
            return 0

        lax.fori_loop(0, CHUNK // 16, grp, 0)
        pltpu.sync_copy(vew, aew.at[colv], add=True)
        pltpu.sync_copy(vsf, asf.at[colv], add=True)
        return 0

    lax.fori_loop(0, NCH, chunk, 0)
    plsc.subcore_barrier()
    pltpu.sync_copy(aew.at[pl.ds(sid * RPT, RPT)], zb)
    pltpu.sync_copy(zb, out_hbm.at[pl.ds(2 * cid * NP + sid * RPT, RPT)])
    pltpu.sync_copy(asf.at[pl.ds(sid * RPT, RPT)], zb)
    pltpu.sync_copy(zb, out_hbm.at[pl.ds((2 * cid + 1) * NP + sid * RPT, RPT)])


def _make_prop(feat):
    @functools.partial(
        pl.kernel,
        mesh=_MESH,
        compiler_params=pltpu.CompilerParams(use_tc_tiling_on_sc=False),
        out_type=jax.ShapeDtypeStruct((NC, NP, feat), _f32),
        scratch_types=[
            pltpu.VMEM((CHUNK,), _i32),
            pltpu.VMEM((CHUNK,), _i32),
            pltpu.VMEM((CHUNK, feat), _f32),
            pltpu.VMEM((ZROWS, feat), _f32),
            pltpu.VMEM_SHARED((NP, feat), _f32),
            pltpu.SemaphoreType.DMA,
        ],
    )
    def _prop(row_hbm, col_hbm, hs_hbm, out_hbm,
              rowv, colv, rows, zb, acc, sem):
        cid = lax.axis_index("c")
        sid = lax.axis_index("s")
        wid = sid * NC + cid

        z16 = jnp.zeros((16,), _f32)
        for zr in range(ZROWS):
            for zc in range(feat // 16):
                zb[zr, pl.ds(zc * 16, 16)] = z16
        for k in range(RPT // ZROWS):
            pltpu.sync_copy(zb, acc.at[pl.ds(sid * RPT + k * ZROWS, ZROWS)])
        rem = RPT % ZROWS
        if rem:
            pltpu.sync_copy(zb.at[pl.ds(0, rem)],
                            acc.at[pl.ds(sid * RPT + RPT - rem, rem)])
        plsc.subcore_barrier()

        def chunk(j, _):
            base = wid * EPT + j * CHUNK
            pltpu.sync_copy(row_hbm.at[pl.ds(base, CHUNK)], rowv)
            pltpu.sync_copy(col_hbm.at[pl.ds(base, CHUNK)], colv)
            pltpu.async_copy(hs_hbm.at[rowv], rows, sem).wait()
            pltpu.sync_copy(rows, acc.at[colv], add=True)
            return 0

        lax.fori_loop(0, NCH, chunk, 0)
        plsc.subcore_barrier()
        for k in range(RPT // ZROWS):
            pltpu.sync_copy(acc.at[pl.ds(sid * RPT + k * ZROWS, ZROWS)], zb)
            pltpu.sync_copy(zb, out_hbm.at[cid, pl.ds(sid * RPT + k * ZROWS,
                                                      ZROWS)])
        if RPT % ZROWS:
            rem = RPT % ZROWS
            off = sid * RPT + RPT - rem
            pltpu.sync_copy(acc.at[pl.ds(off, rem)], zb.at[pl.ds(0, rem)])
            pltpu.sync_copy(zb.at[pl.ds(0, rem)], out_hbm.at[cid, pl.ds(off, rem)])

    return _prop


_prop128 = _make_prop(F)
_prop48 = _make_prop(F3P)


def _tc_prep_body(hist_ref, x_ref, w_ref, dis_ref, c0_ref, h_ref, hs_ref):
    hist = hist_ref[...]
    degn = (hist[0:N] + hist[2 * NP:2 * NP + N]).reshape(N, 1)
    selfc = (hist[NP:NP + N] + hist[3 * NP:3 * NP + N]).reshape(N, 1)
    dis = lax.rsqrt(degn + 1.0)
    c0 = (1.0 - selfc) * dis * dis
    h = jnp.dot(x_ref[...], w_ref[...], preferred_element_type=_f32)
    dis_ref[...] = dis
    c0_ref[...] = c0
    h_ref[...] = h
    hs_ref[...] = dis * h


_tc_prep = pl.pallas_call(
    _tc_prep_body,
    out_shape=[
        jax.ShapeDtypeStruct((N, 1), _f32),
        jax.ShapeDtypeStruct((N, 1), _f32),
        jax.ShapeDtypeStruct((N, F), _f32),
        jax.ShapeDtypeStruct((N, F), _f32),
    ],
)


def _tc_layer_body(p_ref, h_ref, dis_ref, c0_ref, b_ref, g_ref, be_ref,
                   w_ref, hn_ref, hsn_ref):
    p = p_ref[...]
    dis = dis_ref[...]
    u = (dis * (p[0, :N] + p[1, :N]) + c0_ref[...] * h_ref[...] + b_ref[...])
    mu = jnp.mean(u, axis=0, keepdims=True)
    d = u - mu
    var = jnp.mean(d * d, axis=0, keepdims=True)
    v = jnp.maximum(d * lax.rsqrt(var + 1e-5) * g_ref[...] + be_ref[...], 0.0)
    hn = jnp.dot(v, w_ref[...], preferred_element_type=_f32)
    hn_ref[...] = hn
    hsn_ref[...] = dis * hn


def _make_tc_layer(fout):
    return pl.pallas_call(
        _tc_layer_body,
        out_shape=[
            jax.ShapeDtypeStruct((N, fout), _f32),
            jax.ShapeDtypeStruct((N, fout), _f32),
        ],
    )


_tc_layer1 = _make_tc_layer(F)
_tc_layer2 = _make_tc_layer(F3P)


def _tc_final_body(p_ref, h_ref, dis_ref, c0_ref, b_ref, out_ref):
    p = p_ref[...]
    u = (dis_ref[...] * (p[0, :N] + p[1, :N])
         + c0_ref[...] * h_ref[...] + b_ref[...])
    out_ref[...] = u[:, :F3]


_tc_final = pl.pallas_call(
    _tc_final_body,
    out_shape=jax.ShapeDtypeStruct((N, F3), _f32),
)


def kernel(x, edge_idx, W1, b1, g1, be1, W2, b2, g2, be2, W3, b3):
    row = edge_idx[0]
    col = edge_idx[1]
    pad = EPAD - E
    rowp = jnp.concatenate([row, jnp.zeros((pad,), _i32)])
    colp = jnp.concatenate([col, jnp.full((pad,), N, _i32)])

    W3p = jnp.pad(W3, ((0, 0), (0, F3P - F3)))
    b3p = jnp.pad(b3, (0, F3P - F3)).reshape(1, F3P)

    hist = _hist(rowp, colp)
    dis, c0, h1, hs1 = _tc_prep(hist, x, W1)

    p1 = _prop128(rowp, colp, hs1)
    h2, hs2 = _tc_layer1(p1, h1, dis, c0, b1.reshape(1, F),
                         g1.reshape(1, F), be1.reshape(1, F), W2)

    p2 = _prop128(rowp, colp, hs2)
    h3, hs3 = _tc_layer2(p2, h2, dis, c0, b2.reshape(1, F),
                         g2.reshape(1, F), be2.reshape(1, F), W3p)

    p3 = _prop48(rowp, colp, hs3)
    return _tc_final(p3, h3, dis, c0, b3p)


# speedup vs baseline: 18.6573x; 1.2663x over previous
import functools

import jax
import jax.numpy as jnp
from jax import lax
from jax.experimental import pallas as pl
from jax.experimental.pallas import tpu as pltpu
from jax.experimental.pallas import tpu_sc as plsc

N = 10000
E = 320000
F = 128
F3 = 40
F3P = 48

NC = 2
NS = 16
NW = NC * NS

CHUNK = 128
NCH = 79
EPT = NCH * CHUNK
EPAD = NW * EPT
RPT = 632
ZROWS = 32
NP = NS * RPT

_f32 = jnp.float32
_i32 = jnp.int32

_MESH = plsc.VectorSubcoreMesh(core_axis_name="c", subcore_axis_name="s")


@functools.partial(
    pl.kernel,
    mesh=_MESH,
    compiler_params=pltpu.CompilerParams(use_tc_tiling_on_sc=False),
    out_type=jax.ShapeDtypeStruct((2 * NC * NP,), _f32),
    scratch_types=[
        pltpu.VMEM((NCH, CHUNK), _i32),
        pltpu.VMEM((NCH, CHUNK), _i32),
        pltpu.VMEM((CHUNK,), _f32),
        pltpu.VMEM((CHUNK,), _f32),
        pltpu.VMEM((RPT,), _f32),
        pltpu.VMEM_SHARED((NP,), _f32),
        pltpu.VMEM_SHARED((NP,), _f32),
        pltpu.SemaphoreType.DMA,
    ],
)
def _hist(row_hbm, col_hbm, out_hbm, rowv, colv, vew, vsf, zb, aew, asf, semi):
    cid = lax.axis_index("c")
    sid = lax.axis_index("s")
    wid = sid * NC + cid

    ir = pltpu.async_copy(row_hbm.at[wid], rowv, semi)
    ic = pltpu.async_copy(col_hbm.at[wid], colv, semi)

    z16 = jnp.zeros((16,), _f32)

    def zb_body(g, _):
        zb[pl.ds(g * 16, 16)] = z16
        return 0

    lax.fori_loop(0, RPT // 16, zb_body, 0)
    zb[pl.ds(RPT - 16, 16)] = z16
    pltpu.sync_copy(zb, aew.at[pl.ds(sid * RPT, RPT)])
    pltpu.sync_copy(zb, asf.at[pl.ds(sid * RPT, RPT)])
    ir.wait()
    ic.wait()
    plsc.subcore_barrier()

    def chunk(j, _):
        def grp(g, _):
            r = rowv[j, pl.ds(g * 16, 16)]
            c = colv[j, pl.ds(g * 16, 16)]
            ew = jnp.where(r != c, 1.0, 0.0).astype(_f32)
            vew[pl.ds(g * 16, 16)] = ew
            vsf[pl.ds(g * 16, 16)] = 1.0 - ew
            return 0

        lax.fori_loop(0, CHUNK // 16, grp, 0)
        pltpu.sync_copy(vew, aew.at[colv.at[j]], add=True)
        pltpu.sync_copy(vsf, asf.at[colv.at[j]], add=True)
        return 0

    lax.fori_loop(0, NCH, chunk, 0)
    plsc.subcore_barrier()
    pltpu.sync_copy(aew.at[pl.ds(sid * RPT, RPT)], zb)
    pltpu.sync_copy(zb, out_hbm.at[pl.ds(2 * cid * NP + sid * RPT, RPT)])
    pltpu.sync_copy(asf.at[pl.ds(sid * RPT, RPT)], zb)
    pltpu.sync_copy(zb, out_hbm.at[pl.ds((2 * cid + 1) * NP + sid * RPT, RPT)])


def _make_prop(feat):
    @functools.partial(
        pl.kernel,
        mesh=_MESH,
        compiler_params=pltpu.CompilerParams(use_tc_tiling_on_sc=False),
        out_type=jax.ShapeDtypeStruct((NC, NP, feat), _f32),
        scratch_types=[
            pltpu.VMEM((NCH, CHUNK), _i32),
            pltpu.VMEM((NCH, CHUNK), _i32),
            pltpu.VMEM((CHUNK, feat), _f32),
            pltpu.VMEM((ZROWS, feat), _f32),
            pltpu.VMEM_SHARED((NP, feat), _f32),
            pltpu.SemaphoreType.DMA,
            pltpu.SemaphoreType.DMA,
            pltpu.SemaphoreType.DMA,
        ],
    )
    def _prop(row_hbm, col_hbm, hs_hbm, out_hbm,
              rowv, colv, rows0, zb, acc, semi, semg, semz):
        cid = lax.axis_index("c")
        sid = lax.axis_index("s")
        wid = sid * NC + cid

        ir = pltpu.async_copy(row_hbm.at[wid], rowv, semi)
        ic = pltpu.async_copy(col_hbm.at[wid], colv, semi)

        z16 = jnp.zeros((16,), _f32)
        for zr in range(ZROWS):
            for zc in range(feat // 16):
                zb[zr, pl.ds(zc * 16, 16)] = z16
        nz = RPT // ZROWS
        zcopies = [pltpu.async_copy(
            zb, acc.at[pl.ds(sid * RPT + k * ZROWS, ZROWS)], semz)
            for k in range(nz)]
        rem = RPT % ZROWS
        if rem:
            zcopies.append(pltpu.async_copy(
                zb.at[pl.ds(0, rem)],
                acc.at[pl.ds(sid * RPT + RPT - rem, rem)], semz))
        for c in zcopies:
            c.wait()
        ir.wait()
        ic.wait()
        plsc.subcore_barrier()

        def chunk(j, _):
            g = pltpu.async_copy(hs_hbm.at[rowv.at[j]], rows0, semg)
            g.wait()
            pltpu.sync_copy(rows0, acc.at[colv.at[j]], add=True)
            return 0

        lax.fori_loop(0, NCH, chunk, 0)

        plsc.subcore_barrier()
        for k in range(RPT // ZROWS):
            pltpu.sync_copy(acc.at[pl.ds(sid * RPT + k * ZROWS, ZROWS)], zb)
            pltpu.sync_copy(zb, out_hbm.at[cid, pl.ds(sid * RPT + k * ZROWS,
                                                      ZROWS)])
        if RPT % ZROWS:
            rem = RPT % ZROWS
            off = sid * RPT + RPT - rem
            pltpu.sync_copy(acc.at[pl.ds(off, rem)], zb.at[pl.ds(0, rem)])
            pltpu.sync_copy(zb.at[pl.ds(0, rem)], out_hbm.at[cid, pl.ds(off, rem)])

    return _prop


_prop128 = _make_prop(F)
_prop48 = _make_prop(F3P)


def _tc_prep_body(hist_ref, x_ref, w_ref, dis_ref, c0_ref, h_ref, hs_ref):
    hist = hist_ref[...]
    degn = (hist[0:N] + hist[2 * NP:2 * NP + N]).reshape(N, 1)
    selfc = (hist[NP:NP + N] + hist[3 * NP:3 * NP + N]).reshape(N, 1)
    dis = lax.rsqrt(degn + 1.0)
    c0 = (1.0 - selfc) * dis * dis
    h = jnp.dot(x_ref[...], w_ref[...], preferred_element_type=_f32)
    dis_ref[...] = dis
    c0_ref[...] = c0
    h_ref[...] = h
    hs_ref[...] = dis * h


_tc_prep = pl.pallas_call(
    _tc_prep_body,
    out_shape=[
        jax.ShapeDtypeStruct((N, 1), _f32),
        jax.ShapeDtypeStruct((N, 1), _f32),
        jax.ShapeDtypeStruct((N, F), _f32),
        jax.ShapeDtypeStruct((N, F), _f32),
    ],
)


def _tc_layer_body(p_ref, h_ref, dis_ref, c0_ref, b_ref, g_ref, be_ref,
                   w_ref, hn_ref, hsn_ref):
    p = p_ref[...]
    dis = dis_ref[...]
    u = (dis * (p[0, :N] + p[1, :N]) + c0_ref[...] * h_ref[...] + b_ref[...])
    mu = jnp.mean(u, axis=0, keepdims=True)
    d = u - mu
    var = jnp.mean(d * d, axis=0, keepdims=True)
    v = jnp.maximum(d * lax.rsqrt(var + 1e-5) * g_ref[...] + be_ref[...], 0.0)
    hn = jnp.dot(v, w_ref[...], preferred_element_type=_f32)
    hn_ref[...] = hn
    hsn_ref[...] = dis * hn


def _make_tc_layer(fout):
    return pl.pallas_call(
        _tc_layer_body,
        out_shape=[
            jax.ShapeDtypeStruct((N, fout), _f32),
            jax.ShapeDtypeStruct((N, fout), _f32),
        ],
    )


_tc_layer1 = _make_tc_layer(F)
_tc_layer2 = _make_tc_layer(F3P)


def _tc_final_body(p_ref, h_ref, dis_ref, c0_ref, b_ref, out_ref):
    p = p_ref[...]
    u = (dis_ref[...] * (p[0, :N] + p[1, :N])
         + c0_ref[...] * h_ref[...] + b_ref[...])
    out_ref[...] = u[:, :F3]


_tc_final = pl.pallas_call(
    _tc_final_body,
    out_shape=jax.ShapeDtypeStruct((N, F3), _f32),
)


def kernel(x, edge_idx, W1, b1, g1, be1, W2, b2, g2, be2, W3, b3):
    row = edge_idx[0]
    col = edge_idx[1]
    pad = EPAD - E
    rowp = jnp.concatenate([row, jnp.zeros((pad,), _i32)]).reshape(
        NW, NCH, CHUNK)
    colp = jnp.concatenate([col, jnp.full((pad,), N, _i32)]).reshape(
        NW, NCH, CHUNK)

    W3p = jnp.pad(W3, ((0, 0), (0, F3P - F3)))
    b3p = jnp.pad(b3, (0, F3P - F3)).reshape(1, F3P)

    hist = _hist(rowp, colp)
    dis, c0, h1, hs1 = _tc_prep(hist, x, W1)

    p1 = _prop128(rowp, colp, hs1)
    h2, hs2 = _tc_layer1(p1, h1, dis, c0, b1.reshape(1, F),
                         g1.reshape(1, F), be1.reshape(1, F), W2)

    p2 = _prop128(rowp, colp, hs2)
    h3, hs3 = _tc_layer2(p2, h2, dis, c0, b2.reshape(1, F),
                         g2.reshape(1, F), be2.reshape(1, F), W3p)

    p3 = _prop48(rowp, colp, hs3)
    return _tc_final(p3, h3, dis, c0, b3p)
